# baseline (speedup 1.0000x reference)
import functools

import jax
import jax.numpy as jnp
from jax import lax
from jax.experimental import pallas as pl
from jax.experimental.pallas import tpu as pltpu
from jax.experimental.pallas import tpu_sc as plsc

_NJ = 256
_FG_MAX = 64
_FG_THRESH = 0.5
_BG_KEY_OFFSET = 32768.0


def _keep_body(G, M, R, gt_s, x1r, y1r, x2r, y2r,
               keep_ref, fgt_ref, aidx_ref, key_sc):
    f32 = jnp.float32
    x1v = x1r[0]
    y1v = y1r[0]
    x2v = x2r[0]
    y2v = y2r[0]
    area1 = (x2v - x1v + 1.0) * (y2v - y1v + 1.0)

    def g_step(g, carry):
        best, bidx = carry
        gx1 = gt_s[0, 0, g]
        gy1 = gt_s[0, 0, 64 + g]
        gx2 = gt_s[0, 0, 128 + g]
        gy2 = gt_s[0, 0, 192 + g]
        iw = jnp.maximum(jnp.minimum(x2v, gx2) - jnp.maximum(x1v, gx1) + 1.0, 0.0)
        ih = jnp.maximum(jnp.minimum(y2v, gy2) - jnp.maximum(y1v, gy1) + 1.0, 0.0)
        inter = iw * ih
        area2 = (gx2 - gx1 + 1.0) * (gy2 - gy1 + 1.0)
        iou = inter / (area1 + area2 - inter)
        upd = iou > best
        best = jnp.where(upd, iou, best)
        bidx = jnp.where(upd, g.astype(f32), bidx)
        return best, bidx

    mo, aidxf = lax.fori_loop(
        0, G, g_step,
        (jnp.full((R, 128), -1.0, f32), jnp.zeros((R, 128), f32)))
    aidx_ref[0] = aidxf

    ridx = lax.broadcasted_iota(jnp.int32, (R, 128), 0)
    cidx = lax.broadcasted_iota(jnp.int32, (R, 128), 1)
    valid = (ridx * 128 + cidx) < M
    fg = (mo >= _FG_THRESH) & valid
    bg = (mo < _FG_THRESH) & (mo >= 0.0) & valid
    fgf = fg.astype(f32)
    bgf = bg.astype(f32)

    li = lax.broadcasted_iota(jnp.int32, (128, 128), 0)
    lj = lax.broadcasted_iota(jnp.int32, (128, 128), 1)
    tri_inc = (li <= lj).astype(f32)
    ri = lax.broadcasted_iota(jnp.int32, (R, R), 0)
    rj = lax.broadcasted_iota(jnp.int32, (R, R), 1)
    tri_exc = (rj < ri).astype(f32)

    def ranks(maskf):
        csr = lax.dot_general(maskf, tri_inc, (((1,), (0,)), ((), ())),
                              preferred_element_type=f32)
        rowsum = csr[:, 127:128]
        offs = lax.dot_general(tri_exc, rowsum, (((1,), (0,)), ((), ())),
                               preferred_element_type=f32)
        return csr + offs - 1.0

    rank_fg = ranks(fgf)
    rank_bg = ranks(bgf)
    fgn_i = jnp.sum(fgf).astype(jnp.int32)
    bgn_i = jnp.sum(bgf).astype(jnp.int32)

    key = jnp.where(fg, rank_fg, jnp.where(bg, rank_bg + _BG_KEY_OFFSET, -5.0))
    key_sc[...] = key

    fg_this = jnp.where(
        fgn_i > 0,
        jnp.where(bgn_i > 0, jnp.minimum(jnp.int32(_FG_MAX), fgn_i),
                  jnp.int32(_NJ)),
        jnp.int32(0))
    jj = lax.broadcasted_iota(jnp.int32, (_NJ, 1), 0)
    tfg = lax.rem(jj, jnp.maximum(fgn_i, 1))
    tbg = lax.rem(jj - fg_this, jnp.maximum(bgn_i, 1))
    is_fg_slot = jj < fg_this
    tt = jnp.where(is_fg_slot, tfg,
                   tbg + jnp.int32(int(_BG_KEY_OFFSET))).astype(f32)

    laneiota = lax.broadcasted_iota(jnp.int32, (1, 128), 1).astype(f32)

    def r_step(r, acc):
        keyrow = key_sc[pl.ds(r, 1), :]
        sel = tt == keyrow
        mrow = laneiota + r.astype(f32) * 128.0
        return jnp.where(sel, mrow, acc)

    acc = lax.fori_loop(0, R, r_step, jnp.zeros((_NJ, 128), f32))
    keep = jnp.sum(acc, axis=1, keepdims=True)
    keep_ref[0] = keep
    fgt_ref[0] = jnp.full((1, 128), 1.0, f32) * fg_this.astype(f32)


def _gather_body(B, t0, t1, t2, t3, t4, rjg_hbm, o0, o1, o2, o3, o4,
                 idx0, idx1, rows_v, sem):
    wid = lax.axis_index("s") * 2 + lax.axis_index("c")

    @pl.when(wid < B)
    def _():
        pltpu.sync_copy(rjg_hbm.at[wid, pl.ds(0, 128)], idx0)
        pltpu.sync_copy(rjg_hbm.at[wid, pl.ds(128, 128)], idx1)
        for tab, out in ((t0, o0), (t1, o1), (t2, o2), (t3, o3), (t4, o4)):
            for h, idx in ((0, idx0), (1, idx1)):
                pltpu.async_copy(tab.at[idx], rows_v, sem).wait()
                pltpu.sync_copy(rows_v, out.at[wid, pl.ds(h * 128, 128)])


def _out_body(fgt_s, keep_r, rx1, ry1, rx2, ry2, ra, gtm, out_ref):
    f32 = jnp.float32
    keep = keep_r[0]
    rjf = jnp.floor(keep * (1.0 / 128.0))
    cj = keep - rjf * 128.0
    laneiota = lax.broadcasted_iota(jnp.int32, (1, 128), 1).astype(f32)
    lanesel = cj == laneiota

    def lanepick(ref):
        return jnp.sum(jnp.where(lanesel, ref[0], 0.0), axis=1, keepdims=True)

    sx1 = lanepick(rx1)
    sy1 = lanepick(ry1)
    sx2 = lanepick(rx2)
    sy2 = lanepick(ry2)
    sa = lanepick(ra)
    fg_this = fgt_s[0, 0, 0].astype(jnp.int32)
    jj = lax.broadcasted_iota(jnp.int32, (_NJ, 1), 0)
    is_fg_slot = jj < fg_this

    giota = lax.broadcasted_iota(jnp.int32, (1, 64), 1).astype(f32)
    onehot = (sa == giota).astype(f32)
    gtr = lax.dot_general(onehot, gtm[0], (((1,), (1,)), ((), ())),
                          precision=lax.Precision.HIGHEST,
                          preferred_element_type=f32)
    gx1 = gtr[:, 0:1]
    gy1 = gtr[:, 1:2]
    gx2 = gtr[:, 2:3]
    gy2 = gtr[:, 3:4]
    gcls = gtr[:, 4:5]

    ex_w = sx2 - sx1 + 1.0
    ex_h = sy2 - sy1 + 1.0
    ex_cx = sx1 + 0.5 * ex_w
    ex_cy = sy1 + 0.5 * ex_h
    gt_w = gx2 - gx1 + 1.0
    gt_h = gy2 - gy1 + 1.0
    gt_cx = gx1 + 0.5 * gt_w
    gt_cy = gy1 + 0.5 * gt_h
    dx = (gt_cx - ex_cx) / ex_w
    dy = (gt_cy - ex_cy) / ex_h
    dw = jnp.log(gt_w / ex_w)
    dh = jnp.log(gt_h / ex_h)

    labelz = jnp.where(is_fg_slot, gcls, 0.0)
    fgm = labelz > 0.0
    tx = jnp.where(fgm, dx / 0.1, 0.0)
    ty = jnp.where(fgm, dy / 0.1, 0.0)
    tw = jnp.where(fgm, dw / 0.2, 0.0)
    th = jnp.where(fgm, dh / 0.2, 0.0)
    inw = jnp.where(fgm, 1.0, 0.0)

    outm = jnp.concatenate(
        [sx1, sy1, sx2, sy2, labelz, tx, ty, tw, th,
         inw, inw, inw, inw, inw, inw, inw, inw,
         jnp.zeros((_NJ, 7), f32)], axis=1)
    out_ref[0] = outm


def kernel(all_rois, gt_boxes, num_boxes):
    B, N, _ = all_rois.shape
    G = gt_boxes.shape[1]
    M = N + G
    R = -(-M // 128)
    Mpad = R * 128

    coords = jnp.concatenate([all_rois[:, :, 1:5], gt_boxes[:, :, :4]], axis=1)
    coords = jnp.pad(coords, ((0, 0), (0, Mpad - M), (0, 0)))
    ct = coords.transpose(0, 2, 1).reshape(B, 4, R, 128)
    x1, y1, x2, y2 = ct[:, 0], ct[:, 1], ct[:, 2], ct[:, 3]

    gt_t = jnp.swapaxes(gt_boxes, 1, 2)
    gt_pad = jnp.pad(gt_t, ((0, 0), (0, 0), (0, 64 - G)))
    gt_sm = gt_pad.reshape(B, 1, 320)
    gt_mm = jnp.pad(gt_pad, ((0, 0), (0, 3), (0, 0)))

    keep, fgt, aidx = pl.pallas_call(
        functools.partial(_keep_body, G, M, R),
        grid=(B,),
        in_specs=[
            pl.BlockSpec((1, 1, 320), lambda b: (b, 0, 0),
                         memory_space=pltpu.SMEM),
            pl.BlockSpec((1, R, 128), lambda b: (b, 0, 0)),
            pl.BlockSpec((1, R, 128), lambda b: (b, 0, 0)),
            pl.BlockSpec((1, R, 128), lambda b: (b, 0, 0)),
            pl.BlockSpec((1, R, 128), lambda b: (b, 0, 0)),
        ],
        out_specs=[
            pl.BlockSpec((1, _NJ, 1), lambda b: (b, 0, 0)),
            pl.BlockSpec((1, 1, 128), lambda b: (b, 0, 0)),
            pl.BlockSpec((1, R, 128), lambda b: (b, 0, 0)),
        ],
        out_shape=[
            jax.ShapeDtypeStruct((B, _NJ, 1), jnp.float32),
            jax.ShapeDtypeStruct((B, 1, 128), jnp.float32),
            jax.ShapeDtypeStruct((B, R, 128), jnp.float32),
        ],
        scratch_shapes=[
            pltpu.VMEM((R, 128), jnp.float32),
        ],
    )(gt_sm, x1, y1, x2, y2)

    rjg = (keep.reshape(B, _NJ).astype(jnp.int32) // 128
           + jnp.arange(B, dtype=jnp.int32)[:, None] * R)
    tabs = [a.reshape(B * R, 128) for a in (x1, y1, x2, y2, aidx)]

    row_shape = jax.ShapeDtypeStruct((B, _NJ, 128), jnp.float32)
    gx1, gy1, gx2, gy2, ga = pl.kernel(
        functools.partial(_gather_body, B),
        out_type=[row_shape] * 5,
        scratch_types=[
            pltpu.VMEM((128,), jnp.int32),
            pltpu.VMEM((128,), jnp.int32),
            pltpu.VMEM((128, 128), jnp.float32),
            pltpu.SemaphoreType.DMA,
        ],
        mesh=plsc.VectorSubcoreMesh(core_axis_name="c", subcore_axis_name="s"),
    )(tabs[0], tabs[1], tabs[2], tabs[3], tabs[4], rjg)

    out = pl.pallas_call(
        _out_body,
        grid=(B,),
        in_specs=[
            pl.BlockSpec((1, 1, 128), lambda b: (b, 0, 0),
                         memory_space=pltpu.SMEM),
            pl.BlockSpec((1, _NJ, 1), lambda b: (b, 0, 0)),
            pl.BlockSpec((1, _NJ, 128), lambda b: (b, 0, 0)),
            pl.BlockSpec((1, _NJ, 128), lambda b: (b, 0, 0)),
            pl.BlockSpec((1, _NJ, 128), lambda b: (b, 0, 0)),
            pl.BlockSpec((1, _NJ, 128), lambda b: (b, 0, 0)),
            pl.BlockSpec((1, _NJ, 128), lambda b: (b, 0, 0)),
            pl.BlockSpec((1, 8, 64), lambda b: (b, 0, 0)),
        ],
        out_specs=pl.BlockSpec((1, _NJ, 24), lambda b: (b, 0, 0)),
        out_shape=jax.ShapeDtypeStruct((B, _NJ, 24), jnp.float32),
    )(fgt.reshape(B, 1, 128), keep, gx1, gy1, gx2, gy2, ga, gt_mm)

    bcol = jnp.broadcast_to(
        jnp.arange(B, dtype=jnp.float32)[:, None, None], (B, _NJ, 1))
    rois = jnp.concatenate([bcol, out[:, :, 0:4]], axis=2)
    labels = out[:, :, 4]
    targets = out[:, :, 5:9]
    inside_w = out[:, :, 9:13]
    outside_w = out[:, :, 13:17]
    return rois, labels, targets, inside_w, outside_w


# speedup vs baseline: 1.4120x; 1.4120x over previous
import functools

import jax
import jax.numpy as jnp
from jax import lax
from jax.experimental import pallas as pl
from jax.experimental.pallas import tpu as pltpu
from jax.experimental.pallas import tpu_sc as plsc

_NJ = 256
_FG_MAX = 64
_FG_THRESH = 0.5
_BG_KEY_OFFSET = 32768.0


def _keep_body(G, M, R, gt_s, x1r, y1r, x2r, y2r,
               keep_ref, fgt_ref, aidx_ref, key_sc):
    f32 = jnp.float32
    x1v = x1r[0]
    y1v = y1r[0]
    x2v = x2r[0]
    y2v = y2r[0]
    area1 = (x2v - x1v + 1.0) * (y2v - y1v + 1.0)

    def g_step(g, carry):
        best, bidx = carry
        gx1 = gt_s[0, 0, g]
        gy1 = gt_s[0, 0, 64 + g]
        gx2 = gt_s[0, 0, 128 + g]
        gy2 = gt_s[0, 0, 192 + g]
        iw = jnp.maximum(jnp.minimum(x2v, gx2) - jnp.maximum(x1v, gx1) + 1.0, 0.0)
        ih = jnp.maximum(jnp.minimum(y2v, gy2) - jnp.maximum(y1v, gy1) + 1.0, 0.0)
        inter = iw * ih
        area2 = (gx2 - gx1 + 1.0) * (gy2 - gy1 + 1.0)
        iou = inter / (area1 + area2 - inter)
        upd = iou > best
        best = jnp.where(upd, iou, best)
        bidx = jnp.where(upd, g.astype(f32), bidx)
        return best, bidx

    mo, aidxf = lax.fori_loop(
        0, G, g_step,
        (jnp.full((R, 128), -1.0, f32), jnp.zeros((R, 128), f32)))
    aidx_ref[0] = aidxf

    ridx = lax.broadcasted_iota(jnp.int32, (R, 128), 0)
    cidx = lax.broadcasted_iota(jnp.int32, (R, 128), 1)
    valid = (ridx * 128 + cidx) < M
    fg = (mo >= _FG_THRESH) & valid
    bg = (mo < _FG_THRESH) & (mo >= 0.0) & valid
    fgf = fg.astype(f32)
    bgf = bg.astype(f32)

    li = lax.broadcasted_iota(jnp.int32, (128, 128), 0)
    lj = lax.broadcasted_iota(jnp.int32, (128, 128), 1)
    tri_inc = (li <= lj).astype(f32)
    ri = lax.broadcasted_iota(jnp.int32, (R, R), 0)
    rj = lax.broadcasted_iota(jnp.int32, (R, R), 1)
    tri_exc = (rj < ri).astype(f32)

    def ranks(maskf):
        csr = lax.dot_general(maskf, tri_inc, (((1,), (0,)), ((), ())),
                              preferred_element_type=f32)
        rowsum = csr[:, 127:128]
        offs = lax.dot_general(tri_exc, rowsum, (((1,), (0,)), ((), ())),
                               preferred_element_type=f32)
        return csr + offs - 1.0

    rank_fg = ranks(fgf)
    rank_bg = ranks(bgf)
    fgn_i = jnp.sum(fgf).astype(jnp.int32)
    bgn_i = jnp.sum(bgf).astype(jnp.int32)

    key = jnp.where(fg, rank_fg, jnp.where(bg, rank_bg + _BG_KEY_OFFSET, -5.0))
    key_sc[...] = key

    fg_this = jnp.where(
        fgn_i > 0,
        jnp.where(bgn_i > 0, jnp.minimum(jnp.int32(_FG_MAX), fgn_i),
                  jnp.int32(_NJ)),
        jnp.int32(0))
    jj = lax.broadcasted_iota(jnp.int32, (_NJ, 1), 0)
    tfg = lax.rem(jj, jnp.maximum(fgn_i, 1))
    tbg = lax.rem(jj - fg_this, jnp.maximum(bgn_i, 1))
    is_fg_slot = jj < fg_this
    tt = jnp.where(is_fg_slot, tfg,
                   tbg + jnp.int32(int(_BG_KEY_OFFSET))).astype(f32)

    laneiota = lax.broadcasted_iota(jnp.int32, (1, 128), 1).astype(f32)

    def r_step(r, acc):
        keyrow = key_sc[pl.ds(r, 1), :]
        sel = tt == keyrow
        mrow = laneiota + r.astype(f32) * 128.0
        return jnp.where(sel, mrow, acc)

    acc = lax.fori_loop(0, R, r_step, jnp.zeros((_NJ, 128), f32))
    keep = jnp.sum(acc, axis=1, keepdims=True)
    keep_ref[0] = keep
    fgt_ref[0] = jnp.full((1, 128), 1.0, f32) * fg_this.astype(f32)


def _gather_body(B, t0, t1, t2, t3, t4, rjg_hbm, o0, o1, o2, o3, o4,
                 idx0, idx1, buf0, buf1, sem):
    wid = lax.axis_index("s") * 2 + lax.axis_index("c")
    img = wid // 5
    feat = wid - img * 5

    @pl.when(wid < B * 5)
    def _():
        pltpu.sync_copy(rjg_hbm.at[img, pl.ds(0, 128)], idx0)
        pltpu.sync_copy(rjg_hbm.at[img, pl.ds(128, 128)], idx1)
        for f, (tab, out) in enumerate(
                ((t0, o0), (t1, o1), (t2, o2), (t3, o3), (t4, o4))):
            @pl.when(feat == f)
            def _(tab=tab, out=out):
                c0 = pltpu.async_copy(tab.at[idx0], buf0, sem)
                c1 = pltpu.async_copy(tab.at[idx1], buf1, sem)
                c0.wait()
                c1.wait()
                pltpu.sync_copy(buf0, out.at[img, pl.ds(0, 128)])
                pltpu.sync_copy(buf1, out.at[img, pl.ds(128, 128)])


def _out_body(fgt_s, keep_r, rx1, ry1, rx2, ry2, ra, gtm, out_ref):
    f32 = jnp.float32
    keep = keep_r[0]
    rjf = jnp.floor(keep * (1.0 / 128.0))
    cj = keep - rjf * 128.0
    laneiota = lax.broadcasted_iota(jnp.int32, (1, 128), 1).astype(f32)
    lanesel = cj == laneiota

    def lanepick(ref):
        return jnp.sum(jnp.where(lanesel, ref[0], 0.0), axis=1, keepdims=True)

    sx1 = lanepick(rx1)
    sy1 = lanepick(ry1)
    sx2 = lanepick(rx2)
    sy2 = lanepick(ry2)
    sa = lanepick(ra)
    fg_this = fgt_s[0, 0, 0].astype(jnp.int32)
    jj = lax.broadcasted_iota(jnp.int32, (_NJ, 1), 0)
    is_fg_slot = jj < fg_this

    giota = lax.broadcasted_iota(jnp.int32, (1, 64), 1).astype(f32)
    onehot = (sa == giota).astype(f32)
    gtr = lax.dot_general(onehot, gtm[0], (((1,), (1,)), ((), ())),
                          precision=lax.Precision.HIGHEST,
                          preferred_element_type=f32)
    gx1 = gtr[:, 0:1]
    gy1 = gtr[:, 1:2]
    gx2 = gtr[:, 2:3]
    gy2 = gtr[:, 3:4]
    gcls = gtr[:, 4:5]

    ex_w = sx2 - sx1 + 1.0
    ex_h = sy2 - sy1 + 1.0
    ex_cx = sx1 + 0.5 * ex_w
    ex_cy = sy1 + 0.5 * ex_h
    gt_w = gx2 - gx1 + 1.0
    gt_h = gy2 - gy1 + 1.0
    gt_cx = gx1 + 0.5 * gt_w
    gt_cy = gy1 + 0.5 * gt_h
    dx = (gt_cx - ex_cx) / ex_w
    dy = (gt_cy - ex_cy) / ex_h
    dw = jnp.log(gt_w / ex_w)
    dh = jnp.log(gt_h / ex_h)

    labelz = jnp.where(is_fg_slot, gcls, 0.0)
    fgm = labelz > 0.0
    tx = jnp.where(fgm, dx / 0.1, 0.0)
    ty = jnp.where(fgm, dy / 0.1, 0.0)
    tw = jnp.where(fgm, dw / 0.2, 0.0)
    th = jnp.where(fgm, dh / 0.2, 0.0)
    inw = jnp.where(fgm, 1.0, 0.0)

    outm = jnp.concatenate(
        [sx1, sy1, sx2, sy2, labelz, tx, ty, tw, th,
         inw, inw, inw, inw, inw, inw, inw, inw,
         jnp.zeros((_NJ, 7), f32)], axis=1)
    out_ref[0] = outm


def kernel(all_rois, gt_boxes, num_boxes):
    B, N, _ = all_rois.shape
    G = gt_boxes.shape[1]
    M = N + G
    R = -(-M // 128)
    Mpad = R * 128

    coords = jnp.concatenate([all_rois[:, :, 1:5], gt_boxes[:, :, :4]], axis=1)
    coords = jnp.pad(coords, ((0, 0), (0, Mpad - M), (0, 0)))
    ct = coords.transpose(0, 2, 1).reshape(B, 4, R, 128)
    x1, y1, x2, y2 = ct[:, 0], ct[:, 1], ct[:, 2], ct[:, 3]

    gt_t = jnp.swapaxes(gt_boxes, 1, 2)
    gt_pad = jnp.pad(gt_t, ((0, 0), (0, 0), (0, 64 - G)))
    gt_sm = gt_pad.reshape(B, 1, 320)
    gt_mm = jnp.pad(gt_pad, ((0, 0), (0, 3), (0, 0)))

    keep, fgt, aidx = pl.pallas_call(
        functools.partial(_keep_body, G, M, R),
        grid=(B,),
        in_specs=[
            pl.BlockSpec((1, 1, 320), lambda b: (b, 0, 0),
                         memory_space=pltpu.SMEM),
            pl.BlockSpec((1, R, 128), lambda b: (b, 0, 0)),
            pl.BlockSpec((1, R, 128), lambda b: (b, 0, 0)),
            pl.BlockSpec((1, R, 128), lambda b: (b, 0, 0)),
            pl.BlockSpec((1, R, 128), lambda b: (b, 0, 0)),
        ],
        out_specs=[
            pl.BlockSpec((1, _NJ, 1), lambda b: (b, 0, 0)),
            pl.BlockSpec((1, 1, 128), lambda b: (b, 0, 0)),
            pl.BlockSpec((1, R, 128), lambda b: (b, 0, 0)),
        ],
        out_shape=[
            jax.ShapeDtypeStruct((B, _NJ, 1), jnp.float32),
            jax.ShapeDtypeStruct((B, 1, 128), jnp.float32),
            jax.ShapeDtypeStruct((B, R, 128), jnp.float32),
        ],
        scratch_shapes=[
            pltpu.VMEM((R, 128), jnp.float32),
        ],
    )(gt_sm, x1, y1, x2, y2)

    rjg = (keep.reshape(B, _NJ).astype(jnp.int32) // 128
           + jnp.arange(B, dtype=jnp.int32)[:, None] * R)
    tabs = [a.reshape(B * R, 128) for a in (x1, y1, x2, y2, aidx)]

    row_shape = jax.ShapeDtypeStruct((B, _NJ, 128), jnp.float32)
    gx1, gy1, gx2, gy2, ga = pl.kernel(
        functools.partial(_gather_body, B),
        out_type=[row_shape] * 5,
        scratch_types=[
            pltpu.VMEM((128,), jnp.int32),
            pltpu.VMEM((128,), jnp.int32),
            pltpu.VMEM((128, 128), jnp.float32),
            pltpu.VMEM((128, 128), jnp.float32),
            pltpu.SemaphoreType.DMA,
        ],
        mesh=plsc.VectorSubcoreMesh(core_axis_name="c", subcore_axis_name="s"),
    )(tabs[0], tabs[1], tabs[2], tabs[3], tabs[4], rjg)

    out = pl.pallas_call(
        _out_body,
        grid=(B,),
        in_specs=[
            pl.BlockSpec((1, 1, 128), lambda b: (b, 0, 0),
                         memory_space=pltpu.SMEM),
            pl.BlockSpec((1, _NJ, 1), lambda b: (b, 0, 0)),
            pl.BlockSpec((1, _NJ, 128), lambda b: (b, 0, 0)),
            pl.BlockSpec((1, _NJ, 128), lambda b: (b, 0, 0)),
            pl.BlockSpec((1, _NJ, 128), lambda b: (b, 0, 0)),
            pl.BlockSpec((1, _NJ, 128), lambda b: (b, 0, 0)),
            pl.BlockSpec((1, _NJ, 128), lambda b: (b, 0, 0)),
            pl.BlockSpec((1, 8, 64), lambda b: (b, 0, 0)),
        ],
        out_specs=pl.BlockSpec((1, _NJ, 24), lambda b: (b, 0, 0)),
        out_shape=jax.ShapeDtypeStruct((B, _NJ, 24), jnp.float32),
    )(fgt.reshape(B, 1, 128), keep, gx1, gy1, gx2, gy2, ga, gt_mm)

    bcol = jnp.broadcast_to(
        jnp.arange(B, dtype=jnp.float32)[:, None, None], (B, _NJ, 1))
    rois = jnp.concatenate([bcol, out[:, :, 0:4]], axis=2)
    labels = out[:, :, 4]
    targets = out[:, :, 5:9]
    inside_w = out[:, :, 9:13]
    outside_w = out[:, :, 13:17]
    return rois, labels, targets, inside_w, outside_w


# speedup vs baseline: 1.6265x; 1.1519x over previous
import functools

import jax
import jax.numpy as jnp
from jax import lax
from jax.experimental import pallas as pl
from jax.experimental.pallas import tpu as pltpu
from jax.experimental.pallas import tpu_sc as plsc

_NJ = 256
_FG_MAX = 64
_FG_THRESH = 0.5
_BG_KEY_OFFSET = 32768.0


def _keep_body(G, M, R, gt_s, x1r, y1r, x2r, y2r,
               keep_ref, fgt_ref, aidx_ref, key_sc):
    f32 = jnp.float32
    x1v = x1r[0]
    y1v = y1r[0]
    x2v = x2r[0]
    y2v = y2r[0]
    area1 = (x2v - x1v + 1.0) * (y2v - y1v + 1.0)

    def g_step(g, carry):
        best, bidx = carry
        gx1 = gt_s[0, 0, g]
        gy1 = gt_s[0, 0, 64 + g]
        gx2 = gt_s[0, 0, 128 + g]
        gy2 = gt_s[0, 0, 192 + g]
        iw = jnp.maximum(jnp.minimum(x2v, gx2) - jnp.maximum(x1v, gx1) + 1.0, 0.0)
        ih = jnp.maximum(jnp.minimum(y2v, gy2) - jnp.maximum(y1v, gy1) + 1.0, 0.0)
        inter = iw * ih
        area2 = (gx2 - gx1 + 1.0) * (gy2 - gy1 + 1.0)
        iou = inter / (area1 + area2 - inter)
        upd = iou > best
        best = jnp.where(upd, iou, best)
        bidx = jnp.where(upd, g.astype(f32), bidx)
        return best, bidx

    mo, aidxf = lax.fori_loop(
        0, G, g_step,
        (jnp.full((R, 128), -1.0, f32), jnp.zeros((R, 128), f32)),
        unroll=5)
    aidx_ref[0] = aidxf

    ridx = lax.broadcasted_iota(jnp.int32, (R, 128), 0)
    cidx = lax.broadcasted_iota(jnp.int32, (R, 128), 1)
    valid = (ridx * 128 + cidx) < M
    fg = (mo >= _FG_THRESH) & valid
    bg = (mo < _FG_THRESH) & (mo >= 0.0) & valid
    fgf = fg.astype(f32)
    bgf = bg.astype(f32)

    li = lax.broadcasted_iota(jnp.int32, (128, 128), 0)
    lj = lax.broadcasted_iota(jnp.int32, (128, 128), 1)
    tri_inc = (li <= lj).astype(f32)
    ri = lax.broadcasted_iota(jnp.int32, (R, R), 0)
    rj = lax.broadcasted_iota(jnp.int32, (R, R), 1)
    tri_exc = (rj < ri).astype(f32)

    def ranks(maskf):
        csr = lax.dot_general(maskf, tri_inc, (((1,), (0,)), ((), ())),
                              preferred_element_type=f32)
        rowsum = csr[:, 127:128]
        offs = lax.dot_general(tri_exc, rowsum, (((1,), (0,)), ((), ())),
                               preferred_element_type=f32)
        return csr + offs - 1.0

    rank_fg = ranks(fgf)
    rank_bg = ranks(bgf)
    fgn_i = jnp.sum(fgf).astype(jnp.int32)
    bgn_i = jnp.sum(bgf).astype(jnp.int32)

    key = jnp.where(fg, rank_fg, jnp.where(bg, rank_bg + _BG_KEY_OFFSET, -5.0))
    key_sc[...] = key

    fg_this = jnp.where(
        fgn_i > 0,
        jnp.where(bgn_i > 0, jnp.minimum(jnp.int32(_FG_MAX), fgn_i),
                  jnp.int32(_NJ)),
        jnp.int32(0))
    jj = lax.broadcasted_iota(jnp.int32, (_NJ, 1), 0)
    tfg = lax.rem(jj, jnp.maximum(fgn_i, 1))
    tbg = lax.rem(jj - fg_this, jnp.maximum(bgn_i, 1))
    is_fg_slot = jj < fg_this
    tt = jnp.where(is_fg_slot, tfg,
                   tbg + jnp.int32(int(_BG_KEY_OFFSET))).astype(f32)

    laneiota = lax.broadcasted_iota(jnp.int32, (1, 128), 1).astype(f32)

    def r_step(r, acc):
        keyrow = key_sc[pl.ds(r, 1), :]
        sel = tt == keyrow
        mrow = laneiota + r.astype(f32) * 128.0
        return jnp.where(sel, mrow, acc)

    acc = lax.fori_loop(0, R, r_step, jnp.zeros((_NJ, 128), f32), unroll=4)
    keep = jnp.sum(acc, axis=1, keepdims=True)
    keep_ref[0] = keep
    fgt_ref[0] = jnp.full((1, 128), 1.0, f32) * fg_this.astype(f32)


def _gather_body(B, t0, t1, t2, t3, t4, rjg_hbm, o0, o1, o2, o3, o4,
                 idx0, idx1, buf0, buf1, sem):
    wid = lax.axis_index("s") * 2 + lax.axis_index("c")
    img = wid // 5
    feat = wid - img * 5

    @pl.when(wid < B * 5)
    def _():
        pltpu.sync_copy(rjg_hbm.at[img, pl.ds(0, 128)], idx0)
        pltpu.sync_copy(rjg_hbm.at[img, pl.ds(128, 128)], idx1)
        for f, (tab, out) in enumerate(
                ((t0, o0), (t1, o1), (t2, o2), (t3, o3), (t4, o4))):
            @pl.when(feat == f)
            def _(tab=tab, out=out):
                c0 = pltpu.async_copy(tab.at[idx0], buf0, sem)
                c1 = pltpu.async_copy(tab.at[idx1], buf1, sem)
                c0.wait()
                c1.wait()
                pltpu.sync_copy(buf0, out.at[img, pl.ds(0, 128)])
                pltpu.sync_copy(buf1, out.at[img, pl.ds(128, 128)])


def _out_body(fgt_s, keep_r, rx1, ry1, rx2, ry2, ra, gtm, out_ref):
    f32 = jnp.float32
    keep = keep_r[0]
    rjf = jnp.floor(keep * (1.0 / 128.0))
    cj = keep - rjf * 128.0
    laneiota = lax.broadcasted_iota(jnp.int32, (1, 128), 1).astype(f32)
    lanesel = cj == laneiota

    def lanepick(ref):
        return jnp.sum(jnp.where(lanesel, ref[0], 0.0), axis=1, keepdims=True)

    sx1 = lanepick(rx1)
    sy1 = lanepick(ry1)
    sx2 = lanepick(rx2)
    sy2 = lanepick(ry2)
    sa = lanepick(ra)
    fg_this = fgt_s[0, 0, 0].astype(jnp.int32)
    jj = lax.broadcasted_iota(jnp.int32, (_NJ, 1), 0)
    is_fg_slot = jj < fg_this

    giota = lax.broadcasted_iota(jnp.int32, (1, 64), 1).astype(f32)
    onehot = (sa == giota).astype(f32)
    gtr = lax.dot_general(onehot, gtm[0], (((1,), (1,)), ((), ())),
                          precision=lax.Precision.HIGHEST,
                          preferred_element_type=f32)
    gx1 = gtr[:, 0:1]
    gy1 = gtr[:, 1:2]
    gx2 = gtr[:, 2:3]
    gy2 = gtr[:, 3:4]
    gcls = gtr[:, 4:5]

    ex_w = sx2 - sx1 + 1.0
    ex_h = sy2 - sy1 + 1.0
    ex_cx = sx1 + 0.5 * ex_w
    ex_cy = sy1 + 0.5 * ex_h
    gt_w = gx2 - gx1 + 1.0
    gt_h = gy2 - gy1 + 1.0
    gt_cx = gx1 + 0.5 * gt_w
    gt_cy = gy1 + 0.5 * gt_h
    dx = (gt_cx - ex_cx) / ex_w
    dy = (gt_cy - ex_cy) / ex_h
    dw = jnp.log(gt_w / ex_w)
    dh = jnp.log(gt_h / ex_h)

    labelz = jnp.where(is_fg_slot, gcls, 0.0)
    fgm = labelz > 0.0
    tx = jnp.where(fgm, dx / 0.1, 0.0)
    ty = jnp.where(fgm, dy / 0.1, 0.0)
    tw = jnp.where(fgm, dw / 0.2, 0.0)
    th = jnp.where(fgm, dh / 0.2, 0.0)
    inw = jnp.where(fgm, 1.0, 0.0)

    outm = jnp.concatenate(
        [sx1, sy1, sx2, sy2, labelz, tx, ty, tw, th,
         inw, inw, inw, inw, inw, inw, inw, inw,
         jnp.zeros((_NJ, 7), f32)], axis=1)
    out_ref[0] = outm


def kernel(all_rois, gt_boxes, num_boxes):
    B, N, _ = all_rois.shape
    G = gt_boxes.shape[1]
    M = N + G
    R = -(-M // 128)
    Mpad = R * 128

    coords = jnp.concatenate([all_rois[:, :, 1:5], gt_boxes[:, :, :4]], axis=1)
    coords = jnp.pad(coords, ((0, 0), (0, Mpad - M), (0, 0)))
    ct = coords.transpose(0, 2, 1).reshape(B, 4, R, 128)
    x1, y1, x2, y2 = ct[:, 0], ct[:, 1], ct[:, 2], ct[:, 3]

    gt_t = jnp.swapaxes(gt_boxes, 1, 2)
    gt_pad = jnp.pad(gt_t, ((0, 0), (0, 0), (0, 64 - G)))
    gt_sm = gt_pad.reshape(B, 1, 320)
    gt_mm = jnp.pad(gt_pad, ((0, 0), (0, 3), (0, 0)))

    keep, fgt, aidx = pl.pallas_call(
        functools.partial(_keep_body, G, M, R),
        grid=(B,),
        in_specs=[
            pl.BlockSpec((1, 1, 320), lambda b: (b, 0, 0),
                         memory_space=pltpu.SMEM),
            pl.BlockSpec((1, R, 128), lambda b: (b, 0, 0)),
            pl.BlockSpec((1, R, 128), lambda b: (b, 0, 0)),
            pl.BlockSpec((1, R, 128), lambda b: (b, 0, 0)),
            pl.BlockSpec((1, R, 128), lambda b: (b, 0, 0)),
        ],
        out_specs=[
            pl.BlockSpec((1, _NJ, 1), lambda b: (b, 0, 0)),
            pl.BlockSpec((1, 1, 128), lambda b: (b, 0, 0)),
            pl.BlockSpec((1, R, 128), lambda b: (b, 0, 0)),
        ],
        out_shape=[
            jax.ShapeDtypeStruct((B, _NJ, 1), jnp.float32),
            jax.ShapeDtypeStruct((B, 1, 128), jnp.float32),
            jax.ShapeDtypeStruct((B, R, 128), jnp.float32),
        ],
        scratch_shapes=[
            pltpu.VMEM((R, 128), jnp.float32),
        ],
    )(gt_sm, x1, y1, x2, y2)

    rjg = (keep.reshape(B, _NJ).astype(jnp.int32) // 128
           + jnp.arange(B, dtype=jnp.int32)[:, None] * R)
    tabs = [a.reshape(B * R, 128) for a in (x1, y1, x2, y2, aidx)]

    row_shape = jax.ShapeDtypeStruct((B, _NJ, 128), jnp.float32)
    gx1, gy1, gx2, gy2, ga = pl.kernel(
        functools.partial(_gather_body, B),
        out_type=[row_shape] * 5,
        scratch_types=[
            pltpu.VMEM((128,), jnp.int32),
            pltpu.VMEM((128,), jnp.int32),
            pltpu.VMEM((128, 128), jnp.float32),
            pltpu.VMEM((128, 128), jnp.float32),
            pltpu.SemaphoreType.DMA,
        ],
        mesh=plsc.VectorSubcoreMesh(core_axis_name="c", subcore_axis_name="s"),
    )(tabs[0], tabs[1], tabs[2], tabs[3], tabs[4], rjg)

    out = pl.pallas_call(
        _out_body,
        grid=(B,),
        in_specs=[
            pl.BlockSpec((1, 1, 128), lambda b: (b, 0, 0),
                         memory_space=pltpu.SMEM),
            pl.BlockSpec((1, _NJ, 1), lambda b: (b, 0, 0)),
            pl.BlockSpec((1, _NJ, 128), lambda b: (b, 0, 0)),
            pl.BlockSpec((1, _NJ, 128), lambda b: (b, 0, 0)),
            pl.BlockSpec((1, _NJ, 128), lambda b: (b, 0, 0)),
            pl.BlockSpec((1, _NJ, 128), lambda b: (b, 0, 0)),
            pl.BlockSpec((1, _NJ, 128), lambda b: (b, 0, 0)),
            pl.BlockSpec((1, 8, 64), lambda b: (b, 0, 0)),
        ],
        out_specs=pl.BlockSpec((1, _NJ, 24), lambda b: (b, 0, 0)),
        out_shape=jax.ShapeDtypeStruct((B, _NJ, 24), jnp.float32),
    )(fgt.reshape(B, 1, 128), keep, gx1, gy1, gx2, gy2, ga, gt_mm)

    bcol = jnp.broadcast_to(
        jnp.arange(B, dtype=jnp.float32)[:, None, None], (B, _NJ, 1))
    rois = jnp.concatenate([bcol, out[:, :, 0:4]], axis=2)
    labels = out[:, :, 4]
    targets = out[:, :, 5:9]
    inside_w = out[:, :, 9:13]
    outside_w = out[:, :, 13:17]
    return rois, labels, targets, inside_w, outside_w


# speedup vs baseline: 1.6560x; 1.0181x over previous
import functools

import jax
import jax.numpy as jnp
from jax import lax
from jax.experimental import pallas as pl
from jax.experimental.pallas import tpu as pltpu
from jax.experimental.pallas import tpu_sc as plsc

_NJ = 256
_FG_MAX = 64
_FG_THRESH = 0.5
_BG_KEY_OFFSET = 32768.0


def _keep_body(G, M, R, gt_s, x1r, y1r, x2r, y2r,
               keep_ref, fgt_ref, aidx_ref, key_sc):
    f32 = jnp.float32
    x1v = x1r[0]
    y1v = y1r[0]
    x2v = x2r[0]
    y2v = y2r[0]
    area1 = (x2v - x1v + 1.0) * (y2v - y1v + 1.0)

    def g_step(g, carry):
        best, bidx = carry
        gx1 = gt_s[0, 0, g]
        gy1 = gt_s[0, 0, 64 + g]
        gx2 = gt_s[0, 0, 128 + g]
        gy2 = gt_s[0, 0, 192 + g]
        iw = jnp.maximum(jnp.minimum(x2v, gx2) - jnp.maximum(x1v, gx1) + 1.0, 0.0)
        ih = jnp.maximum(jnp.minimum(y2v, gy2) - jnp.maximum(y1v, gy1) + 1.0, 0.0)
        inter = iw * ih
        area2 = (gx2 - gx1 + 1.0) * (gy2 - gy1 + 1.0)
        iou = inter / (area1 + area2 - inter)
        upd = iou > best
        best = jnp.where(upd, iou, best)
        bidx = jnp.where(upd, g.astype(f32), bidx)
        return best, bidx

    mo, aidxf = lax.fori_loop(
        0, G, g_step,
        (jnp.full((R, 128), -1.0, f32), jnp.zeros((R, 128), f32)),
        unroll=10)
    aidx_ref[0] = aidxf

    ridx = lax.broadcasted_iota(jnp.int32, (R, 128), 0)
    cidx = lax.broadcasted_iota(jnp.int32, (R, 128), 1)
    valid = (ridx * 128 + cidx) < M
    fg = (mo >= _FG_THRESH) & valid
    bg = (mo < _FG_THRESH) & (mo >= 0.0) & valid
    fgf = fg.astype(f32)
    bgf = bg.astype(f32)

    li = lax.broadcasted_iota(jnp.int32, (128, 128), 0)
    lj = lax.broadcasted_iota(jnp.int32, (128, 128), 1)
    tri_inc = (li <= lj).astype(f32)
    ri = lax.broadcasted_iota(jnp.int32, (R, R), 0)
    rj = lax.broadcasted_iota(jnp.int32, (R, R), 1)
    tri_exc = (rj < ri).astype(f32)

    def ranks(maskf):
        csr = lax.dot_general(maskf, tri_inc, (((1,), (0,)), ((), ())),
                              preferred_element_type=f32)
        rowsum = csr[:, 127:128]
        offs = lax.dot_general(tri_exc, rowsum, (((1,), (0,)), ((), ())),
                               preferred_element_type=f32)
        return csr + offs - 1.0

    rank_fg = ranks(fgf)
    rank_bg = ranks(bgf)
    fgn_i = jnp.sum(fgf).astype(jnp.int32)
    bgn_i = jnp.sum(bgf).astype(jnp.int32)

    key = jnp.where(fg, rank_fg, jnp.where(bg, rank_bg + _BG_KEY_OFFSET, -5.0))
    key_sc[...] = key

    fg_this = jnp.where(
        fgn_i > 0,
        jnp.where(bgn_i > 0, jnp.minimum(jnp.int32(_FG_MAX), fgn_i),
                  jnp.int32(_NJ)),
        jnp.int32(0))
    jj = lax.broadcasted_iota(jnp.int32, (_NJ, 1), 0)
    tfg = lax.rem(jj, jnp.maximum(fgn_i, 1))
    tbg = lax.rem(jj - fg_this, jnp.maximum(bgn_i, 1))
    is_fg_slot = jj < fg_this
    tt = jnp.where(is_fg_slot, tfg,
                   tbg + jnp.int32(int(_BG_KEY_OFFSET))).astype(f32)

    laneiota = lax.broadcasted_iota(jnp.int32, (1, 128), 1).astype(f32)

    def r_step(r, acc):
        keyrow = key_sc[pl.ds(r, 1), :]
        sel = tt == keyrow
        mrow = laneiota + r.astype(f32) * 128.0
        return jnp.where(sel, mrow, acc)

    acc = lax.fori_loop(0, R, r_step, jnp.zeros((_NJ, 128), f32), unroll=8)
    keep = jnp.sum(acc, axis=1, keepdims=True)
    keep_ref[0] = keep
    fgt_ref[0] = jnp.full((1, 128), 1.0, f32) * fg_this.astype(f32)


def _gather_body(B, t0, t1, t2, t3, t4, rjg_hbm, o0, o1, o2, o3, o4,
                 idx0, idx1, buf0, buf1, sem):
    wid = lax.axis_index("s") * 2 + lax.axis_index("c")
    img = wid // 5
    feat = wid - img * 5

    @pl.when(wid < B * 5)
    def _():
        pltpu.sync_copy(rjg_hbm.at[img, pl.ds(0, 128)], idx0)
        pltpu.sync_copy(rjg_hbm.at[img, pl.ds(128, 128)], idx1)
        for f, (tab, out) in enumerate(
                ((t0, o0), (t1, o1), (t2, o2), (t3, o3), (t4, o4))):
            @pl.when(feat == f)
            def _(tab=tab, out=out):
                c0 = pltpu.async_copy(tab.at[idx0], buf0, sem)
                c1 = pltpu.async_copy(tab.at[idx1], buf1, sem)
                c0.wait()
                c1.wait()
                pltpu.sync_copy(buf0, out.at[img, pl.ds(0, 128)])
                pltpu.sync_copy(buf1, out.at[img, pl.ds(128, 128)])


def _out_body(fgt_s, keep_r, rx1, ry1, rx2, ry2, ra, gtm, out_ref):
    f32 = jnp.float32
    keep = keep_r[0]
    rjf = jnp.floor(keep * (1.0 / 128.0))
    cj = keep - rjf * 128.0
    laneiota = lax.broadcasted_iota(jnp.int32, (1, 128), 1).astype(f32)
    lanesel = cj == laneiota

    def lanepick(ref):
        return jnp.sum(jnp.where(lanesel, ref[0], 0.0), axis=1, keepdims=True)

    sx1 = lanepick(rx1)
    sy1 = lanepick(ry1)
    sx2 = lanepick(rx2)
    sy2 = lanepick(ry2)
    sa = lanepick(ra)
    fg_this = fgt_s[0, 0, 0].astype(jnp.int32)
    jj = lax.broadcasted_iota(jnp.int32, (_NJ, 1), 0)
    is_fg_slot = jj < fg_this

    giota = lax.broadcasted_iota(jnp.int32, (1, 64), 1).astype(f32)
    onehot = (sa == giota).astype(f32)
    gtr = lax.dot_general(onehot, gtm[0], (((1,), (1,)), ((), ())),
                          precision=lax.Precision.HIGHEST,
                          preferred_element_type=f32)
    gx1 = gtr[:, 0:1]
    gy1 = gtr[:, 1:2]
    gx2 = gtr[:, 2:3]
    gy2 = gtr[:, 3:4]
    gcls = gtr[:, 4:5]

    ex_w = sx2 - sx1 + 1.0
    ex_h = sy2 - sy1 + 1.0
    ex_cx = sx1 + 0.5 * ex_w
    ex_cy = sy1 + 0.5 * ex_h
    gt_w = gx2 - gx1 + 1.0
    gt_h = gy2 - gy1 + 1.0
    gt_cx = gx1 + 0.5 * gt_w
    gt_cy = gy1 + 0.5 * gt_h
    dx = (gt_cx - ex_cx) / ex_w
    dy = (gt_cy - ex_cy) / ex_h
    dw = jnp.log(gt_w / ex_w)
    dh = jnp.log(gt_h / ex_h)

    labelz = jnp.where(is_fg_slot, gcls, 0.0)
    fgm = labelz > 0.0
    tx = jnp.where(fgm, dx / 0.1, 0.0)
    ty = jnp.where(fgm, dy / 0.1, 0.0)
    tw = jnp.where(fgm, dw / 0.2, 0.0)
    th = jnp.where(fgm, dh / 0.2, 0.0)
    inw = jnp.where(fgm, 1.0, 0.0)

    outm = jnp.concatenate(
        [sx1, sy1, sx2, sy2, labelz, tx, ty, tw, th,
         inw, inw, inw, inw, inw, inw, inw, inw,
         jnp.zeros((_NJ, 7), f32)], axis=1)
    out_ref[0] = outm


def kernel(all_rois, gt_boxes, num_boxes):
    B, N, _ = all_rois.shape
    G = gt_boxes.shape[1]
    M = N + G
    R = -(-M // 128)
    Mpad = R * 128

    coords = jnp.concatenate([all_rois[:, :, 1:5], gt_boxes[:, :, :4]], axis=1)
    coords = jnp.pad(coords, ((0, 0), (0, Mpad - M), (0, 0)))
    ct = coords.transpose(0, 2, 1).reshape(B, 4, R, 128)
    x1, y1, x2, y2 = ct[:, 0], ct[:, 1], ct[:, 2], ct[:, 3]

    gt_t = jnp.swapaxes(gt_boxes, 1, 2)
    gt_pad = jnp.pad(gt_t, ((0, 0), (0, 0), (0, 64 - G)))
    gt_sm = gt_pad.reshape(B, 1, 320)
    gt_mm = jnp.pad(gt_pad, ((0, 0), (0, 3), (0, 0)))

    keep, fgt, aidx = pl.pallas_call(
        functools.partial(_keep_body, G, M, R),
        grid=(B,),
        in_specs=[
            pl.BlockSpec((1, 1, 320), lambda b: (b, 0, 0),
                         memory_space=pltpu.SMEM),
            pl.BlockSpec((1, R, 128), lambda b: (b, 0, 0)),
            pl.BlockSpec((1, R, 128), lambda b: (b, 0, 0)),
            pl.BlockSpec((1, R, 128), lambda b: (b, 0, 0)),
            pl.BlockSpec((1, R, 128), lambda b: (b, 0, 0)),
        ],
        out_specs=[
            pl.BlockSpec((1, _NJ, 1), lambda b: (b, 0, 0)),
            pl.BlockSpec((1, 1, 128), lambda b: (b, 0, 0)),
            pl.BlockSpec((1, R, 128), lambda b: (b, 0, 0)),
        ],
        out_shape=[
            jax.ShapeDtypeStruct((B, _NJ, 1), jnp.float32),
            jax.ShapeDtypeStruct((B, 1, 128), jnp.float32),
            jax.ShapeDtypeStruct((B, R, 128), jnp.float32),
        ],
        scratch_shapes=[
            pltpu.VMEM((R, 128), jnp.float32),
        ],
    )(gt_sm, x1, y1, x2, y2)

    rjg = (keep.reshape(B, _NJ).astype(jnp.int32) // 128
           + jnp.arange(B, dtype=jnp.int32)[:, None] * R)
    tabs = [a.reshape(B * R, 128) for a in (x1, y1, x2, y2, aidx)]

    row_shape = jax.ShapeDtypeStruct((B, _NJ, 128), jnp.float32)
    gx1, gy1, gx2, gy2, ga = pl.kernel(
        functools.partial(_gather_body, B),
        out_type=[row_shape] * 5,
        scratch_types=[
            pltpu.VMEM((128,), jnp.int32),
            pltpu.VMEM((128,), jnp.int32),
            pltpu.VMEM((128, 128), jnp.float32),
            pltpu.VMEM((128, 128), jnp.float32),
            pltpu.SemaphoreType.DMA,
        ],
        mesh=plsc.VectorSubcoreMesh(core_axis_name="c", subcore_axis_name="s"),
    )(tabs[0], tabs[1], tabs[2], tabs[3], tabs[4], rjg)

    out = pl.pallas_call(
        _out_body,
        grid=(B,),
        in_specs=[
            pl.BlockSpec((1, 1, 128), lambda b: (b, 0, 0),
                         memory_space=pltpu.SMEM),
            pl.BlockSpec((1, _NJ, 1), lambda b: (b, 0, 0)),
            pl.BlockSpec((1, _NJ, 128), lambda b: (b, 0, 0)),
            pl.BlockSpec((1, _NJ, 128), lambda b: (b, 0, 0)),
            pl.BlockSpec((1, _NJ, 128), lambda b: (b, 0, 0)),
            pl.BlockSpec((1, _NJ, 128), lambda b: (b, 0, 0)),
            pl.BlockSpec((1, _NJ, 128), lambda b: (b, 0, 0)),
            pl.BlockSpec((1, 8, 64), lambda b: (b, 0, 0)),
        ],
        out_specs=pl.BlockSpec((1, _NJ, 24), lambda b: (b, 0, 0)),
        out_shape=jax.ShapeDtypeStruct((B, _NJ, 24), jnp.float32),
    )(fgt.reshape(B, 1, 128), keep, gx1, gy1, gx2, gy2, ga, gt_mm)

    bcol = jnp.broadcast_to(
        jnp.arange(B, dtype=jnp.float32)[:, None, None], (B, _NJ, 1))
    rois = jnp.concatenate([bcol, out[:, :, 0:4]], axis=2)
    labels = out[:, :, 4]
    targets = out[:, :, 5:9]
    inside_w = out[:, :, 9:13]
    outside_w = out[:, :, 13:17]
    return rois, labels, targets, inside_w, outside_w
